# baseline (speedup 1.0000x reference)
import functools

import jax
import jax.numpy as jnp
from jax import lax
from jax.experimental import pallas as pl
from jax.experimental.pallas import tpu as pltpu
from jax.experimental.pallas import tpu_sc as plsc

_VOCAB = 100000
_PAD_ID = 0
_B = 1024
_R = 16
_NC = 2
_NS = 16
_NW = _NC * _NS
_RPW = _B // _NW


def _tc_fill_body(scal_ref, trg_ref, out_ref):
    base = scal_ref[0]
    trg = trg_ref[...]
    r = trg.shape[0]
    col = jax.lax.broadcasted_iota(jnp.int32, (r, _VOCAB), 1)
    out_ref[...] = jnp.where((col == _PAD_ID) | (trg == _PAD_ID), 0.0, base)


def _tc_fill(scal, trg):
    return pl.pallas_call(
        _tc_fill_body,
        grid=(_B // _R,),
        in_specs=[
            pl.BlockSpec(memory_space=pltpu.SMEM),
            pl.BlockSpec((_R, 1), lambda i: (i, 0)),
        ],
        out_specs=pl.BlockSpec((_R, _VOCAB), lambda i: (i, 0)),
        out_shape=jax.ShapeDtypeStruct((_B, _VOCAB), jnp.float32),
    )(scal, trg)


def _sc_scatter_body(out_hbm, trg_hbm, conf_hbm, trg_v, conf_v, idx_v, val_v, sem):
    wid = lax.axis_index("s") * _NC + lax.axis_index("c")
    rbase = wid * _RPW

    pltpu.sync_copy(trg_hbm.at[pl.ds(rbase, _RPW)], trg_v)
    pltpu.sync_copy(conf_hbm, conf_v)
    cvec = conf_v[...]
    lanes = lax.iota(jnp.int32, 16)
    for j in range(_RPW // 16):
        tvec = trg_v[pl.ds(j * 16, 16)]
        rvec = lanes + (rbase + j * 16)
        idx_v[pl.ds(j * 16, 16)] = rvec * _VOCAB + tvec
        val_v[pl.ds(j * 16, 16)] = jnp.where(tvec == _PAD_ID, 0.0, cvec)
    pltpu.async_copy(val_v, out_hbm.at[idx_v], sem).wait()


_sc_scatter = functools.partial(
    pl.kernel,
    out_type=(),
    mesh=plsc.VectorSubcoreMesh(core_axis_name="c", subcore_axis_name="s"),
    scratch_types=[
        pltpu.VMEM((_RPW,), jnp.int32),
        pltpu.VMEM((16,), jnp.float32),
        pltpu.VMEM((_RPW,), jnp.int32),
        pltpu.VMEM((_RPW,), jnp.float32),
        pltpu.SemaphoreType.DMA,
    ],
)(_sc_scatter_body)


def kernel(trg_token_ids_batch, confidence, smoothing_value):
    b = trg_token_ids_batch.shape[0]
    base = (smoothing_value / (_VOCAB - 2)).astype(jnp.float32)
    scal = jnp.stack([base, base])
    filled = _tc_fill(scal, trg_token_ids_batch)
    out_ref = jax.new_ref(filled.reshape(b * _VOCAB))
    conf16 = jnp.full((16,), confidence, jnp.float32)
    _sc_scatter(out_ref, trg_token_ids_batch.reshape(b), conf16)
    return out_ref[...].reshape(b, _VOCAB)
